# baseline (speedup 1.0000x reference)
import functools

import jax
import jax.numpy as jnp
from jax.experimental import pallas as pl
from jax.experimental.pallas import tpu as pltpu


def _kt_kernel(wk_ref, x_ref, kt_ref):
    kt_ref[...] = jax.lax.dot_general(
        wk_ref[...], x_ref[...],
        dimension_numbers=(((1,), (1,)), ((), ())),
        preferred_element_type=jnp.float32)


def _main_kernel(x_ref, wq_ref, bq_ref, bk_ref, kt_ref, out_ref,
                 qb_ref, ab_ref, *, topk, scale, nblk):
    i = pl.program_id(0)
    rb, d = x_ref.shape
    n = kt_ref.shape[1]
    cw = 256
    nchunks = n // cw

    qb_w = qb_ref.at[i % 2]
    qb_r = qb_ref.at[(i + 1) % 2]
    ab_w = ab_ref.at[(i + 1) % 2]

    a_prev = ab_ref[i % 2]

    x_blk = x_ref[...]
    q_prev = qb_r[...]
    qbk = jnp.sum(q_prev * bk_ref[...], axis=1, keepdims=True)

    @pl.when(i < nblk)
    def _():
        for j in range(nchunks):
            sl = pl.ds(j * cw, cw)
            qc = jax.lax.dot_general(
                x_blk, wq_ref[sl, :],
                dimension_numbers=(((1,), (1,)), ((), ())),
                preferred_element_type=jnp.float32)
            qb_w[:, sl] = qc + bq_ref[:, sl]

    @pl.when((i >= 1) & (i <= nblk))
    def _():
        for j in range(nchunks):
            sl = pl.ds(j * cw, cw)
            ac = jax.lax.dot_general(
                q_prev, kt_ref[:, sl],
                dimension_numbers=(((1,), (0,)), ((), ())),
                preferred_element_type=jnp.float32)
            ac = (ac + qbk) / scale
            ab_w[:, sl] = jnp.where(ac >= 0.0, ac, 0.2 * ac)

    w = n // 16
    vsl = [a_prev[:, v * w:(v + 1) * w] for v in range(16)]
    hs = [jnp.maximum(vsl[2 * t], vsl[2 * t + 1]) for t in range(8)]
    ls = [jnp.minimum(vsl[2 * t], vsl[2 * t + 1]) for t in range(8)]
    while len(hs) > 1:
        nh, nl = [], []
        for t in range(0, len(hs), 2):
            h1, h2, l1, l2 = hs[t], hs[t + 1], ls[t], ls[t + 1]
            lw = jnp.where(h1 >= h2, l1, l2)
            nh.append(jnp.maximum(h1, h2))
            nl.append(jnp.maximum(jnp.minimum(h1, h2), lw))
        hs, ls = nh, nl
    cand = jnp.concatenate([hs[0], ls[0]], axis=1)

    m = jnp.full((rb, 1), jnp.inf, jnp.float32)
    for j in range(topk):
        m = jnp.max(jnp.where(cand < m, cand, -jnp.inf),
                    axis=1, keepdims=True)
        if j == 0:
            m1 = m
    t2 = m

    c = jnp.sum(jnp.where(a_prev >= t2, 1.0, 0.0), axis=1, keepdims=True)
    e_x = c - float(topk)
    max_e = jnp.max(e_x)

    def peel_up(m_cur, thr, k):
        m_nxt = jnp.min(jnp.where(a_prev > m_cur, a_prev, jnp.inf),
                        axis=1, keepdims=True)
        return m_nxt, jnp.where(e_x >= k, m_nxt, thr)

    m_cur, thr = peel_up(t2, t2, 1.0)
    m_cur, thr = peel_up(m_cur, thr, 2.0)

    def up_cond(carry):
        k, _, _ = carry
        return k <= max_e

    def up_body(carry):
        k, m_cur, thr = carry
        m_nxt, thr = peel_up(m_cur, thr, k)
        return k + 1.0, m_nxt, thr

    _, _, thr = jax.lax.while_loop(up_cond, up_body, (3.0, m_cur, thr))
    s = jnp.where((a_prev >= thr) & (a_prev != 0.0), a_prev, -1e9)
    e = jnp.exp(s - m1)
    out_ref[...] = e / jnp.sum(e, axis=1, keepdims=True)


def kernel(x, Wq, bq, Wk, bk):
    B, N, D = x.shape
    TOPK = 32
    x0 = x.reshape(N, D)
    RB = min(256, N)
    NB = N // RB

    kt = pl.pallas_call(
        _kt_kernel,
        grid=(D // RB,),
        in_specs=[
            pl.BlockSpec((RB, D), lambda i: (i, 0)),
            pl.BlockSpec((N, D), lambda i: (0, 0)),
        ],
        out_specs=pl.BlockSpec((RB, N), lambda i: (i, 0)),
        out_shape=jax.ShapeDtypeStruct((D, N), jnp.float32),
    )(Wk, x0)

    out = pl.pallas_call(
        functools.partial(_main_kernel, topk=TOPK, scale=D ** 0.5, nblk=NB),
        grid=(NB + 2,),
        in_specs=[
            pl.BlockSpec((RB, D), lambda i: (jnp.minimum(i, NB - 1), 0)),
            pl.BlockSpec((D, D), lambda i: (0, 0)),
            pl.BlockSpec((1, D), lambda i: (0, 0)),
            pl.BlockSpec((1, D), lambda i: (0, 0)),
            pl.BlockSpec((D, N), lambda i: (0, 0)),
        ],
        out_specs=pl.BlockSpec((RB, N), lambda i: (jnp.maximum(i - 2, 0), 0)),
        out_shape=jax.ShapeDtypeStruct((N, N), jnp.float32),
        scratch_shapes=[
            pltpu.VMEM((2, RB, D), jnp.float32),
            pltpu.VMEM((2, RB, N), jnp.float32),
        ],
    )(x0, Wq, bq.reshape(1, D), bk.reshape(1, D), kt)

    return out.reshape(B, N, N)


# speedup vs baseline: 1.0931x; 1.0931x over previous
import functools

import jax
import jax.numpy as jnp
from jax.experimental import pallas as pl
from jax.experimental.pallas import tpu as pltpu


def _kt_kernel(wk_ref, x_ref, kt_ref):
    kt_ref[...] = jax.lax.dot_general(
        wk_ref[...], x_ref[...],
        dimension_numbers=(((1,), (1,)), ((), ())),
        preferred_element_type=jnp.float32)


def _main_kernel(x_ref, wq_ref, bq_ref, bk_ref, kt_ref, out_ref,
                 qb_ref, ab_ref, *, topk, scale, nblk):
    i = pl.program_id(0)
    rb, d = x_ref.shape
    n = kt_ref.shape[1]
    cw = 256
    nchunks = n // cw

    qb_w = qb_ref.at[i % 2]
    qb_r = qb_ref.at[(i + 1) % 2]
    ab_w = ab_ref.at[(i + 1) % 2]

    a_prev = ab_ref[i % 2]

    x_blk = x_ref[...]
    q_prev = qb_r[...]
    qbk = jnp.sum(q_prev * bk_ref[...], axis=1, keepdims=True)

    for j in range(nchunks):
        sl = pl.ds(j * cw, cw)
        qc = jax.lax.dot_general(
            x_blk, wq_ref[sl, :],
            dimension_numbers=(((1,), (1,)), ((), ())),
            preferred_element_type=jnp.float32)
        qb_w[:, sl] = qc + bq_ref[:, sl]
    for j in range(nchunks):
        sl = pl.ds(j * cw, cw)
        ac = jax.lax.dot_general(
            q_prev, kt_ref[:, sl],
            dimension_numbers=(((1,), (0,)), ((), ())),
            preferred_element_type=jnp.float32)
        ac = (ac + qbk) / scale
        ab_w[:, sl] = jnp.where(ac >= 0.0, ac, 0.2 * ac)

    w = n // 16
    vsl = [a_prev[:, v * w:(v + 1) * w] for v in range(16)]
    hs = [jnp.maximum(vsl[2 * t], vsl[2 * t + 1]) for t in range(8)]
    ls = [jnp.minimum(vsl[2 * t], vsl[2 * t + 1]) for t in range(8)]
    while len(hs) > 1:
        nh, nl = [], []
        for t in range(0, len(hs), 2):
            h1, h2, l1, l2 = hs[t], hs[t + 1], ls[t], ls[t + 1]
            lw = jnp.where(h1 >= h2, l1, l2)
            nh.append(jnp.maximum(h1, h2))
            nl.append(jnp.maximum(jnp.minimum(h1, h2), lw))
        hs, ls = nh, nl
    cand = jnp.concatenate([hs[0], ls[0]], axis=1)

    m = jnp.full((rb, 1), jnp.inf, jnp.float32)
    for j in range(topk):
        m = jnp.max(jnp.where(cand < m, cand, -jnp.inf),
                    axis=1, keepdims=True)
        if j == 0:
            m1 = m
    t2 = m

    c = jnp.sum(jnp.where(a_prev >= t2, 1.0, 0.0), axis=1, keepdims=True)
    e_x = c - float(topk)
    max_e = jnp.max(e_x)

    def peel_up(m_cur, thr, k):
        m_nxt = jnp.min(jnp.where(a_prev > m_cur, a_prev, jnp.inf),
                        axis=1, keepdims=True)
        return m_nxt, jnp.where(e_x >= k, m_nxt, thr)

    m_cur, thr = peel_up(t2, t2, 1.0)
    m_cur, thr = peel_up(m_cur, thr, 2.0)

    def up_cond(carry):
        k, _, _ = carry
        return k <= max_e

    def up_body(carry):
        k, m_cur, thr = carry
        m_nxt, thr = peel_up(m_cur, thr, k)
        return k + 1.0, m_nxt, thr

    _, _, thr = jax.lax.while_loop(up_cond, up_body, (3.0, m_cur, thr))
    s = jnp.where((a_prev >= thr) & (a_prev != 0.0), a_prev, -1e9)
    e = jnp.exp(s - m1)
    out_ref[...] = e / jnp.sum(e, axis=1, keepdims=True)


def kernel(x, Wq, bq, Wk, bk):
    B, N, D = x.shape
    TOPK = 32
    x0 = x.reshape(N, D)
    RB = min(256, N)
    NB = N // RB

    kt = pl.pallas_call(
        _kt_kernel,
        grid=(D // RB,),
        in_specs=[
            pl.BlockSpec((RB, D), lambda i: (i, 0)),
            pl.BlockSpec((N, D), lambda i: (0, 0)),
        ],
        out_specs=pl.BlockSpec((RB, N), lambda i: (i, 0)),
        out_shape=jax.ShapeDtypeStruct((D, N), jnp.float32),
    )(Wk, x0)

    out = pl.pallas_call(
        functools.partial(_main_kernel, topk=TOPK, scale=D ** 0.5, nblk=NB),
        grid=(NB + 2,),
        in_specs=[
            pl.BlockSpec((RB, D), lambda i: (jnp.minimum(i, NB - 1), 0)),
            pl.BlockSpec((D, D), lambda i: (0, 0)),
            pl.BlockSpec((1, D), lambda i: (0, 0)),
            pl.BlockSpec((1, D), lambda i: (0, 0)),
            pl.BlockSpec((D, N), lambda i: (0, 0)),
        ],
        out_specs=pl.BlockSpec((RB, N), lambda i: (jnp.maximum(i - 2, 0), 0)),
        out_shape=jax.ShapeDtypeStruct((N, N), jnp.float32),
        scratch_shapes=[
            pltpu.VMEM((2, RB, D), jnp.float32),
            pltpu.VMEM((2, RB, N), jnp.float32),
        ],
    )(x0, Wq, bq.reshape(1, D), bk.reshape(1, D), kt)

    return out.reshape(B, N, N)
